# baseline (speedup 1.0000x reference)
import functools

import jax
import jax.numpy as jnp
from jax import lax
from jax.experimental import pallas as pl
from jax.experimental.pallas import tpu as pltpu
from jax.experimental.pallas import tpu_sc as plsc

P_PAD = 24
NW = 32
CHUNK = 128


def _dot(x, y):
    return jax.lax.dot_general(
        x.astype(jnp.bfloat16), y.astype(jnp.bfloat16),
        (((1,), (0,)), ((), ())), preferred_element_type=jnp.float32)


def _tc_body(feat_ref, scal_ref, W1p_ref, b1_ref, W2_ref, b2_ref,
             W3a_ref, W3b_ref, b3_ref, W4_ref, b4_ref,
             slW1_ref, slb1_ref, slW2_ref, slb2_ref, out_ref, *, B):
    BP = B * P_PAD
    a = feat_ref[...]
    vmrep = _dot(a[:, 6:7], jnp.ones((1, 256), jnp.float32))
    h1 = jnp.maximum(_dot(a, W1p_ref[...]) + b1_ref[...], 0.0)
    h = (_dot(h1, W2_ref[...]) + b2_ref[...]) * vmrep
    pooled = jnp.max(h.reshape(B, P_PAD, 256), axis=1)
    g2 = _dot(pooled, W3b_ref[...]) + b3_ref[...]
    t1 = _dot(h, W3a_ref[...])
    h3 = jnp.maximum(t1.reshape(B, P_PAD, 256) + g2[:, None, :], 0.0)
    h3 = h3.reshape(BP, 256)
    h4 = (_dot(h3, W4_ref[...]) + b4_ref[...]) * vmrep[:, :128]
    x_poly = jnp.max(h4.reshape(B, P_PAD, 128), axis=1)

    s = scal_ref[...]
    sl1 = jnp.maximum(_dot(s[:, 0:1], slW1_ref[...]) + slb1_ref[...], 0.0)
    sl = _dot(sl1, slW2_ref[...]) + slb2_ref[...]
    hs = _dot(s[:, 1:2], jnp.ones((1, 128), jnp.float32))
    out_ref[...] = x_poly + hs * sl


def _sc_body(tc_hbm, idx_hbm, table_hbm, out_hbm, idx_v, acc_v, sem):
    n = tc_hbm.shape[0]
    rows = n // NW
    wid = lax.axis_index("s") * 2 + lax.axis_index("c")
    base = wid * rows
    for c in range(rows // CHUNK):
        off = base + c * CHUNK
        pltpu.sync_copy(tc_hbm.at[pl.ds(off, CHUNK)], acc_v)
        pltpu.sync_copy(idx_hbm.at[:, pl.ds(off, CHUNK)], idx_v)
        cps = [pltpu.async_copy(table_hbm.at[idx_v.at[j]], acc_v, sem,
                                add=True) for j in range(4)]
        for cp in cps:
            cp.wait()
        pltpu.sync_copy(acc_v, out_hbm.at[pl.ds(off, CHUNK)])


def kernel(polygon_center, polygon_type, polygon_on_route, polygon_tl_status,
           polygon_has_speed_limit, polygon_speed_limit, point_position,
           point_vector, point_orientation, valid_mask,
           pe_W1, pe_b1, pe_W2, pe_b2, pe_W3, pe_b3, pe_W4, pe_b4,
           sl_W1, sl_b1, sl_W2, sl_b2, type_emb, on_route_emb, tl_emb,
           unknown_speed_emb):
    bs, M, P = point_orientation.shape[0], point_orientation.shape[1], point_orientation.shape[3]
    N = bs * M
    B = 512

    bf = jnp.bfloat16
    pt_pos = (point_position[:, :, 0]
              - polygon_center[..., None, :2]).astype(bf)
    vec = point_vector[:, :, 0].astype(bf)
    ori = point_orientation[:, :, 0]
    trig = jnp.stack([jnp.cos(ori), jnp.sin(ori)], axis=-1).astype(bf)
    vmz = jnp.stack([valid_mask.astype(bf),
                     jnp.zeros(valid_mask.shape, bf)], axis=-1)
    rows = jnp.concatenate([pt_pos, vec, trig, vmz], axis=-1)
    pad = P_PAD - P
    feat = jnp.concatenate(
        [rows, jnp.zeros((bs, M, pad, 8), bf)], axis=2).reshape(N * P_PAD, 8)

    scal = jnp.concatenate(
        [polygon_speed_limit[..., None].astype(bf),
         polygon_has_speed_limit[..., None].astype(bf),
         jnp.zeros((bs, M, 6), bf)], axis=-1)
    scal = scal.reshape(N, 8)

    W1p = jnp.zeros((8, 128), jnp.float32).at[:6].set(pe_W1)
    W3a, W3b = pe_W3[:256], pe_W3[256:]
    row = lambda b: b.reshape(1, -1)

    ctable = jnp.concatenate(
        [type_emb, on_route_emb, tl_emb, unknown_speed_emb,
         jnp.zeros((7, 128), jnp.float32)], axis=0)
    idx_all = jnp.stack(
        [polygon_type.reshape(N),
         3 + polygon_on_route.reshape(N),
         5 + polygon_tl_status.reshape(N),
         jnp.where(polygon_has_speed_limit.reshape(N), 10, 9)],
        axis=0).astype(jnp.int32)

    grid = N // B
    const = lambda shape: pl.BlockSpec(shape, lambda i: (0, 0))
    out = pl.pallas_call(
        functools.partial(_tc_body, B=B),
        grid=(grid,),
        in_specs=[
            pl.BlockSpec((B * P_PAD, 8), lambda i: (i, 0)),
            pl.BlockSpec((B, 8), lambda i: (i, 0)),
            const((8, 128)), const((1, 128)),
            const((128, 256)), const((1, 256)),
            const((256, 256)), const((256, 256)), const((1, 256)),
            const((256, 128)), const((1, 128)),
            const((1, 128)), const((1, 128)),
            const((128, 128)), const((1, 128)),
        ],
        out_specs=pl.BlockSpec((B, 128), lambda i: (i, 0)),
        out_shape=jax.ShapeDtypeStruct((N, 128), jnp.float32),
    )(feat, scal, W1p, row(pe_b1), pe_W2, row(pe_b2),
      W3a, W3b, row(pe_b3), pe_W4, row(pe_b4),
      sl_W1, row(sl_b1), sl_W2, row(sl_b2))

    sc_k = functools.partial(
        pl.kernel,
        out_type=jax.ShapeDtypeStruct((N, 128), jnp.float32),
        mesh=plsc.VectorSubcoreMesh(core_axis_name="c",
                                    subcore_axis_name="s"),
        scratch_types=[
            pltpu.VMEM((4, CHUNK), jnp.int32),
            pltpu.VMEM((CHUNK, 128), jnp.float32),
            pltpu.SemaphoreType.DMA,
        ],
    )(_sc_body)
    final = sc_k(out, idx_all, ctable)
    return final.reshape(bs, M, 128)
